# baseline (speedup 1.0000x reference)
import functools

import jax
import jax.numpy as jnp
from jax import lax
from jax.experimental import pallas as pl
from jax.experimental.pallas import tpu as pltpu
from jax.experimental.pallas import tpu_sc as plsc

BATCH = 16384
NF = 32
NC, NS = 2, 16
NW = NC * NS
B_PER_W = BATCH // NW


def _sc_gather(P, Q, U, V, user_id, item_id):
    mesh = plsc.VectorSubcoreMesh(core_axis_name="c", subcore_axis_name="s")
    row = jax.ShapeDtypeStruct((BATCH, NF), jnp.float32)

    @functools.partial(
        pl.kernel,
        mesh=mesh,
        out_type=(row, row, row, row),
        compiler_params=pltpu.CompilerParams(use_tc_tiling_on_sc=False),
        scratch_types=[
            pltpu.VMEM((B_PER_W,), jnp.int32),
            pltpu.VMEM((B_PER_W,), jnp.int32),
            pltpu.VMEM((B_PER_W, NF), jnp.float32),
            pltpu.VMEM((B_PER_W, NF), jnp.float32),
            pltpu.VMEM((B_PER_W, NF), jnp.float32),
            pltpu.VMEM((B_PER_W, NF), jnp.float32),
            pltpu.SemaphoreType.DMA,
        ],
    )
    def k(p_hbm, q_hbm, u_hbm, v_hbm, iu_hbm, ii_hbm,
          pmf_hbm, qmf_hbm, pml_hbm, qml_hbm,
          iu_v, ii_v, pv, qv, uv, vv, sem):
        wid = lax.axis_index("s") * NC + lax.axis_index("c")
        base = wid * B_PER_W
        pltpu.sync_copy(iu_hbm.at[pl.ds(base, B_PER_W)], iu_v)
        pltpu.sync_copy(ii_hbm.at[pl.ds(base, B_PER_W)], ii_v)
        c1 = pltpu.async_copy(p_hbm.at[iu_v], pv, sem)
        c2 = pltpu.async_copy(q_hbm.at[ii_v], qv, sem)
        c3 = pltpu.async_copy(u_hbm.at[iu_v], uv, sem)
        c4 = pltpu.async_copy(v_hbm.at[ii_v], vv, sem)
        c1.wait()
        c2.wait()
        c3.wait()
        c4.wait()
        pltpu.sync_copy(pv, pmf_hbm.at[pl.ds(base, B_PER_W)])
        pltpu.sync_copy(qv, qmf_hbm.at[pl.ds(base, B_PER_W)])
        pltpu.sync_copy(uv, pml_hbm.at[pl.ds(base, B_PER_W)])
        pltpu.sync_copy(vv, qml_hbm.at[pl.ds(base, B_PER_W)])

    return k(P, Q, U, V, user_id, item_id)


def _tc_mlp_body(pmf_ref, qmf_ref, pml_ref, qml_ref,
                 w0_ref, b0_ref, w1_ref, b1_ref, w2_ref, b2_ref,
                 wp_ref, bp_ref, out_ref):
    f32 = jnp.float32
    h = (jnp.dot(pml_ref[...], w0_ref[:NF, :], preferred_element_type=f32)
         + jnp.dot(qml_ref[...], w0_ref[NF:, :], preferred_element_type=f32)
         + b0_ref[...])
    h = jnp.maximum(h, 0.0)
    h = jnp.dot(h, w1_ref[...], preferred_element_type=f32) + b1_ref[...]
    h = jnp.maximum(h, 0.0)
    h = jnp.dot(h, w2_ref[...], preferred_element_type=f32) + b2_ref[...]
    h = jnp.maximum(h, 0.0)
    gmf = pmf_ref[...] * qmf_ref[...]
    out = (jnp.dot(gmf, wp_ref[:NF, :], preferred_element_type=f32)
           + jnp.dot(h, wp_ref[NF:, :], preferred_element_type=f32)
           + bp_ref[...])
    out_ref[...] = out


def _tc_mlp(pmf, qmf, pml, qml, W0, b0, W1, b1, W2, b2, Wp, bp):
    blk = 2048
    grid = (BATCH // blk,)
    in_row = pl.BlockSpec((blk, NF), lambda i: (i, 0))
    full = lambda a: pl.BlockSpec(a.shape, lambda i: (0,) * a.ndim)
    return pl.pallas_call(
        _tc_mlp_body,
        grid=grid,
        in_specs=[in_row, in_row, in_row, in_row,
                  full(W0), full(b0), full(W1), full(b1),
                  full(W2), full(b2), full(Wp), full(bp)],
        out_specs=pl.BlockSpec((blk, 1), lambda i: (i, 0)),
        out_shape=jax.ShapeDtypeStruct((BATCH, 1), jnp.float32),
    )(pmf, qmf, pml, qml, W0, b0, W1, b1, W2, b2, Wp, bp)


def kernel(user_id, item_id, P, Q, U, V, W0, b0, W1, b1, W2, b2, Wp, bp):
    pmf, qmf, pml, qml = _sc_gather(P, Q, U, V, user_id, item_id)
    out = _tc_mlp(pmf, qmf, pml, qml,
                  W0, b0.reshape(1, -1), W1, b1.reshape(1, -1),
                  W2, b2.reshape(1, -1), Wp, bp.reshape(1, -1))
    return jnp.squeeze(out, axis=1)


# speedup vs baseline: 1.4105x; 1.4105x over previous
import functools

import jax
import jax.numpy as jnp
from jax import lax
from jax.experimental import pallas as pl
from jax.experimental.pallas import tpu as pltpu
from jax.experimental.pallas import tpu_sc as plsc

BATCH = 16384
NF = 32
NC, NS = 2, 16
NW = NC * NS
B_PER_W = BATCH // NW


RB = 128
ROUNDS = B_PER_W // RB


def _sc_gather(P, Q, U, V, user_id, item_id):
    mesh = plsc.VectorSubcoreMesh(core_axis_name="c", subcore_axis_name="s")
    row = jax.ShapeDtypeStruct((BATCH, NF), jnp.float32)

    @functools.partial(
        pl.kernel,
        mesh=mesh,
        out_type=(row, row, row, row),
        compiler_params=pltpu.CompilerParams(needs_layout_passes=False),
        scratch_types=[
            pltpu.VMEM((B_PER_W,), jnp.int32),
            pltpu.VMEM((B_PER_W,), jnp.int32),
            pltpu.VMEM((RB, NF), jnp.float32),
            pltpu.VMEM((RB, NF), jnp.float32),
            pltpu.VMEM((RB, NF), jnp.float32),
            pltpu.VMEM((RB, NF), jnp.float32),
            pltpu.SemaphoreType.DMA,
            pltpu.SemaphoreType.DMA,
            pltpu.SemaphoreType.DMA,
            pltpu.SemaphoreType.DMA,
        ],
    )
    def k(p_hbm, q_hbm, u_hbm, v_hbm, iu_hbm, ii_hbm,
          pmf_hbm, qmf_hbm, pml_hbm, qml_hbm,
          iu_v, ii_v, pv, qv, uv, vv, sp, sq, su, sv):
        wid = lax.axis_index("s") * NC + lax.axis_index("c")
        base = wid * B_PER_W
        pltpu.sync_copy(iu_hbm.at[pl.ds(base, B_PER_W)], iu_v)
        pltpu.sync_copy(ii_hbm.at[pl.ds(base, B_PER_W)], ii_v)
        lane = lax.broadcasted_iota(jnp.int32, (16,), 0)

        @pl.loop(0, ROUNDS)
        def _(r):
            r0 = r * RB

            @pl.loop(0, RB // 16)
            def _(cc):
                cu = iu_v[pl.ds(r0 + cc * 16, 16)]
                ci = ii_v[pl.ds(r0 + cc * 16, 16)]
                for e in range(16):
                    jj = cc * 16 + e
                    u = jnp.max(jnp.where(lane == e, cu, 0))
                    i = jnp.max(jnp.where(lane == e, ci, 0))
                    pltpu.async_copy(
                        p_hbm.at[pl.ds(u, 1)], pv.at[pl.ds(jj, 1)], sp)
                    pltpu.async_copy(
                        q_hbm.at[pl.ds(i, 1)], qv.at[pl.ds(jj, 1)], sq)
                    pltpu.async_copy(
                        u_hbm.at[pl.ds(u, 1)], uv.at[pl.ds(jj, 1)], su)
                    pltpu.async_copy(
                        v_hbm.at[pl.ds(i, 1)], vv.at[pl.ds(jj, 1)], sv)

            @pl.loop(0, RB)
            def _(jj):
                pltpu.make_async_copy(
                    p_hbm.at[pl.ds(0, 1)], pv.at[pl.ds(jj, 1)], sp).wait()
                pltpu.make_async_copy(
                    q_hbm.at[pl.ds(0, 1)], qv.at[pl.ds(jj, 1)], sq).wait()
                pltpu.make_async_copy(
                    u_hbm.at[pl.ds(0, 1)], uv.at[pl.ds(jj, 1)], su).wait()
                pltpu.make_async_copy(
                    v_hbm.at[pl.ds(0, 1)], vv.at[pl.ds(jj, 1)], sv).wait()

            pltpu.sync_copy(pv, pmf_hbm.at[pl.ds(base + r0, RB)])
            pltpu.sync_copy(qv, qmf_hbm.at[pl.ds(base + r0, RB)])
            pltpu.sync_copy(uv, pml_hbm.at[pl.ds(base + r0, RB)])
            pltpu.sync_copy(vv, qml_hbm.at[pl.ds(base + r0, RB)])

    return k(P, Q, U, V, user_id, item_id)


def _tc_mlp_body(pmf_ref, qmf_ref, pml_ref, qml_ref,
                 w0_ref, b0_ref, w1_ref, b1_ref, w2_ref, b2_ref,
                 wp_ref, bp_ref, out_ref):
    f32 = jnp.float32
    h = (jnp.dot(pml_ref[...], w0_ref[:NF, :], preferred_element_type=f32)
         + jnp.dot(qml_ref[...], w0_ref[NF:, :], preferred_element_type=f32)
         + b0_ref[...])
    h = jnp.maximum(h, 0.0)
    h = jnp.dot(h, w1_ref[...], preferred_element_type=f32) + b1_ref[...]
    h = jnp.maximum(h, 0.0)
    h = jnp.dot(h, w2_ref[...], preferred_element_type=f32) + b2_ref[...]
    h = jnp.maximum(h, 0.0)
    gmf = pmf_ref[...] * qmf_ref[...]
    out = (jnp.dot(gmf, wp_ref[:NF, :], preferred_element_type=f32)
           + jnp.dot(h, wp_ref[NF:, :], preferred_element_type=f32)
           + bp_ref[...])
    out_ref[...] = out


def _tc_mlp(pmf, qmf, pml, qml, W0, b0, W1, b1, W2, b2, Wp, bp):
    blk = 2048
    grid = (BATCH // blk,)
    in_row = pl.BlockSpec((blk, NF), lambda i: (i, 0))
    full = lambda a: pl.BlockSpec(a.shape, lambda i: (0,) * a.ndim)
    return pl.pallas_call(
        _tc_mlp_body,
        grid=grid,
        in_specs=[in_row, in_row, in_row, in_row,
                  full(W0), full(b0), full(W1), full(b1),
                  full(W2), full(b2), full(Wp), full(bp)],
        out_specs=pl.BlockSpec((blk, 1), lambda i: (i, 0)),
        out_shape=jax.ShapeDtypeStruct((BATCH, 1), jnp.float32),
    )(pmf, qmf, pml, qml, W0, b0, W1, b1, W2, b2, Wp, bp)


def kernel(user_id, item_id, P, Q, U, V, W0, b0, W1, b1, W2, b2, Wp, bp):
    pmf, qmf, pml, qml = _sc_gather(P, Q, U, V, user_id, item_id)
    out = _tc_mlp(pmf, qmf, pml, qml,
                  W0, b0.reshape(1, -1), W1, b1.reshape(1, -1),
                  W2, b2.reshape(1, -1), Wp, bp.reshape(1, -1))
    return jnp.squeeze(out, axis=1)
